# baseline (speedup 1.0000x reference)
import functools

import jax
import jax.numpy as jnp
from jax import lax
from jax.experimental import pallas as pl
from jax.experimental.pallas import tpu as pltpu
from jax.experimental.pallas import tpu_sc as plsc

BATCH = 16384
FIELDS = 26
OUT = 32
N_ROWS = 1000000
NUM_CORES = 2
NUM_SUBCORES = 16
NW = NUM_CORES * NUM_SUBCORES
B_PER_W = BATCH // NW
CHUNK = 128
BT_PER_W = B_PER_W // CHUNK
N_CHUNKS = FIELDS * BT_PER_W
GROUPS = CHUNK // 16
DEPTH = 8
assert N_CHUNKS % DEPTH == 0 and DEPTH % 2 == 0

_T_BLK = 32768
_T_SUB = _T_BLK // 4
_T_GRID = (N_ROWS + _T_BLK - 1) // _T_BLK
_PACKED_ROWS = _T_GRID * _T_SUB


def _pack_body(x_ref, y_ref):
    x = x_ref[...]
    eye = jnp.eye(OUT, dtype=jnp.float32)
    xt = [lax.dot_general(x[:, k * _T_SUB:(k + 1) * _T_SUB], eye,
                          (((0,), (0,)), ((), ())),
                          precision=lax.Precision.HIGHEST,
                          preferred_element_type=jnp.float32)
          for k in range(4)]
    y_ref[...] = jnp.concatenate(xt, axis=1)


_pack_table = pl.pallas_call(
    _pack_body,
    grid=(_T_GRID,),
    in_specs=[pl.BlockSpec((OUT, _T_BLK), lambda i: (0, i))],
    out_specs=pl.BlockSpec((_T_SUB, 128), lambda i: (i, 0)),
    out_shape=jax.ShapeDtypeStruct((_PACKED_ROWS, 128), jnp.float32),
)


def _rsqrt(x):
    i = lax.bitcast_convert_type(x, jnp.int32)
    i = jnp.full((16,), 0x5F3759DF, jnp.int32) - (i >> 1)
    y = lax.bitcast_convert_type(i, jnp.float32)
    for _ in range(3):
        y = y * (1.5 - 0.5 * x * y * y)
    return y


_mesh = plsc.VectorSubcoreMesh(core_axis_name="c", subcore_axis_name="s")


@functools.partial(
    pl.kernel,
    out_type=jax.ShapeDtypeStruct(
        (FIELDS, OUT // 8, BATCH // 128, 8, 128), jnp.float32),
    mesh=_mesh,
    scratch_types=[
        pltpu.VMEM((FIELDS, BT_PER_W, CHUNK), jnp.int32),
        [pltpu.VMEM((CHUNK, OUT), jnp.float32) for _ in range(DEPTH)],
        [pltpu.VMEM((OUT // 8, 8, CHUNK), jnp.float32) for _ in range(2)],
        [pltpu.SemaphoreType.DMA for _ in range(DEPTH)],
        [pltpu.SemaphoreType.DMA for _ in range(2)],
    ],
    compiler_params=pltpu.CompilerParams(
        needs_layout_passes=False, use_tc_tiling_on_sc=False
    ),
)
def _gather_normalize(table_hbm, idx_hbm, out_hbm,
                      idx_v, gbufs, obufs, gsems, wsems):
    wid = lax.axis_index("s") * NUM_CORES + lax.axis_index("c")

    lanes = lax.iota(jnp.int32, 16)

    def fld_bt(g):
        return g // BT_PER_W, lax.rem(g, BT_PER_W)

    def start_gather(g, p):
        f, bt = fld_bt(g)
        pltpu.async_copy(table_hbm.at[idx_v.at[f, bt]], gbufs[p], gsems[p])

    def wait_gather(g, p):
        f, bt = fld_bt(g)
        pltpu.make_async_copy(table_hbm.at[idx_v.at[f, bt]], gbufs[p],
                              gsems[p]).wait()

    def out_slice(g):
        f, bt = fld_bt(g)
        return out_hbm.at[f, :, wid * BT_PER_W + bt]

    def start_write(g, q):
        pltpu.async_copy(obufs[q], out_slice(g), wsems[q])

    def wait_write(g, q):
        pltpu.make_async_copy(obufs[q], out_slice(g), wsems[q]).wait()

    def normalize(p, q):
        src_v, dst_v = gbufs[p], obufs[q]

        def group_body(gr, c):
            row_ids = gr * 16 + lanes
            cols = [(lanes + j) & 31 for j in range(OUT)]
            elems = [plsc.load_gather(src_v, [row_ids, cols[j]])
                     for j in range(OUT)]
            acc = jnp.zeros((16,), jnp.float32)
            for e in elems:
                acc = acc + e * e
            inv = _rsqrt(acc)
            for j, e in enumerate(elems):
                plsc.store_scatter(
                    dst_v, [cols[j] >> 3, cols[j] & 7, row_ids], e * inv)
            return c

        lax.fori_loop(0, GROUPS, group_body, 0)

    pltpu.sync_copy(idx_hbm.at[wid], idx_v)
    for p in range(DEPTH):
        start_gather(p, p)

    def round_body(t, carry):
        for p in range(DEPTH):
            g = t * DEPTH + p
            q = p % 2
            wait_gather(g, p)

            @pl.when(g >= 2)
            def _():
                wait_write(g - 2, q)

            normalize(p, q)
            start_write(g, q)

            @pl.when(g + DEPTH < N_CHUNKS)
            def _():
                start_gather(g + DEPTH, p)

        return carry

    lax.fori_loop(0, N_CHUNKS // DEPTH, round_body, 0)
    wait_write(N_CHUNKS - 2, 0)
    wait_write(N_CHUNKS - 1, 1)


def kernel(src, categories_means, categories_logvars):
    del categories_logvars
    packed = _pack_table(categories_means.T)
    table = packed.reshape(_PACKED_ROWS * 4, OUT)
    t = src.astype(jnp.int32)
    idx = (t & ~(_T_BLK - 1)) + 4 * (t & (_T_SUB - 1)) \
        + ((t >> _T_SUB.bit_length() - 1) & 3)
    idx = idx.reshape(NW, BT_PER_W, CHUNK, FIELDS).transpose(0, 3, 1, 2)
    x = _gather_normalize(table, idx)
    return x.transpose(2, 4, 0, 1, 3).reshape(BATCH, FIELDS, OUT)


# speedup vs baseline: 2.8858x; 2.8858x over previous
import functools

import jax
import jax.numpy as jnp
from jax import lax
from jax.experimental import pallas as pl
from jax.experimental.pallas import tpu as pltpu
from jax.experimental.pallas import tpu_sc as plsc

BATCH = 16384
FIELDS = 26
OUT = 32
N_ROWS = 1000000
NUM_CORES = 2
NUM_SUBCORES = 16
NW = NUM_CORES * NUM_SUBCORES
B_PER_W = BATCH // NW
CHUNK = 128
BT_PER_W = B_PER_W // CHUNK
N_CHUNKS = FIELDS * BT_PER_W
GROUPS = CHUNK // 16
DEPTH = 8
assert N_CHUNKS % DEPTH == 0 and DEPTH % 2 == 0

_T_BLK = 32768
_T_SUB = _T_BLK // 4
_T_GRID = (N_ROWS + _T_BLK - 1) // _T_BLK
_PACKED_ROWS = _T_GRID * _T_SUB


def _pack_body(x_ref, y_ref):
    x = x_ref[...]
    xs = jnp.concatenate(
        [x[:, k * _T_SUB:(k + 1) * _T_SUB] for k in range(4)], axis=0)
    y_ref[...] = xs.T


_pack_table = pl.pallas_call(
    _pack_body,
    grid=(_T_GRID,),
    in_specs=[pl.BlockSpec((OUT, _T_BLK), lambda i: (0, i))],
    out_specs=pl.BlockSpec((_T_SUB, 128), lambda i: (i, 0)),
    out_shape=jax.ShapeDtypeStruct((_PACKED_ROWS, 128), jnp.float32),
)


def _rsqrt(x):
    i = lax.bitcast_convert_type(x, jnp.int32)
    i = jnp.full((16,), 0x5F3759DF, jnp.int32) - (i >> 1)
    y = lax.bitcast_convert_type(i, jnp.float32)
    for _ in range(3):
        y = y * (1.5 - 0.5 * x * y * y)
    return y


_mesh = plsc.VectorSubcoreMesh(core_axis_name="c", subcore_axis_name="s")


@functools.partial(
    pl.kernel,
    out_type=jax.ShapeDtypeStruct(
        (FIELDS, OUT // 8, BATCH // 128, 8, 128), jnp.float32),
    mesh=_mesh,
    scratch_types=[
        pltpu.VMEM((FIELDS, BT_PER_W, CHUNK), jnp.int32),
        [pltpu.VMEM((CHUNK, OUT), jnp.float32) for _ in range(DEPTH)],
        [pltpu.VMEM((OUT // 8, 8, CHUNK), jnp.float32) for _ in range(2)],
        [pltpu.SemaphoreType.DMA for _ in range(DEPTH)],
        [pltpu.SemaphoreType.DMA for _ in range(2)],
    ],
    compiler_params=pltpu.CompilerParams(
        needs_layout_passes=False, use_tc_tiling_on_sc=False
    ),
)
def _gather_normalize(table_hbm, idx_hbm, out_hbm,
                      idx_v, gbufs, obufs, gsems, wsems):
    wid = lax.axis_index("s") * NUM_CORES + lax.axis_index("c")

    lanes = lax.iota(jnp.int32, 16)

    def fld_bt(g):
        return g // BT_PER_W, lax.rem(g, BT_PER_W)

    def start_gather(g, p):
        f, bt = fld_bt(g)
        pltpu.async_copy(table_hbm.at[idx_v.at[f, bt]], gbufs[p], gsems[p])

    def wait_gather(g, p):
        f, bt = fld_bt(g)
        pltpu.make_async_copy(table_hbm.at[idx_v.at[f, bt]], gbufs[p],
                              gsems[p]).wait()

    def out_slice(g):
        f, bt = fld_bt(g)
        return out_hbm.at[f, :, wid * BT_PER_W + bt]

    def start_write(g, q):
        pltpu.async_copy(obufs[q], out_slice(g), wsems[q])

    def wait_write(g, q):
        pltpu.make_async_copy(obufs[q], out_slice(g), wsems[q]).wait()

    def normalize(p, q):
        src_v, dst_v = gbufs[p], obufs[q]

        def group_body(gr, c):
            row_ids = gr * 16 + lanes
            cols = [(lanes + j) & 31 for j in range(OUT)]
            elems = [plsc.load_gather(src_v, [row_ids, cols[j]])
                     for j in range(OUT)]
            acc = jnp.zeros((16,), jnp.float32)
            for e in elems:
                acc = acc + e * e
            inv = _rsqrt(acc)
            for j, e in enumerate(elems):
                plsc.store_scatter(
                    dst_v, [cols[j] >> 3, cols[j] & 7, row_ids], e * inv)
            return c

        lax.fori_loop(0, GROUPS, group_body, 0)

    pltpu.sync_copy(idx_hbm.at[wid], idx_v)
    for p in range(DEPTH):
        start_gather(p, p)

    def round_body(t, carry):
        for p in range(DEPTH):
            g = t * DEPTH + p
            q = p % 2
            wait_gather(g, p)

            @pl.when(g >= 2)
            def _():
                wait_write(g - 2, q)

            normalize(p, q)
            start_write(g, q)

            @pl.when(g + DEPTH < N_CHUNKS)
            def _():
                start_gather(g + DEPTH, p)

        return carry

    lax.fori_loop(0, N_CHUNKS // DEPTH, round_body, 0)
    wait_write(N_CHUNKS - 2, 0)
    wait_write(N_CHUNKS - 1, 1)


def kernel(src, categories_means, categories_logvars):
    del categories_logvars
    packed = _pack_table(categories_means.T)
    table = packed.reshape(_PACKED_ROWS * 4, OUT)
    t = src.astype(jnp.int32)
    idx = (t & ~(_T_BLK - 1)) + 4 * (t & (_T_SUB - 1)) \
        + ((t >> _T_SUB.bit_length() - 1) & 3)
    idx = idx.reshape(NW, BT_PER_W, CHUNK, FIELDS).transpose(0, 3, 1, 2)
    x = _gather_normalize(table, idx)
    return x.transpose(2, 4, 0, 1, 3).reshape(BATCH, FIELDS, OUT)


# speedup vs baseline: 2.9202x; 1.0119x over previous
import functools

import jax
import jax.numpy as jnp
from jax import lax
from jax.experimental import pallas as pl
from jax.experimental.pallas import tpu as pltpu
from jax.experimental.pallas import tpu_sc as plsc

BATCH = 16384
FIELDS = 26
OUT = 32
N_ROWS = 1000000
NUM_CORES = 2
NUM_SUBCORES = 16
NW = NUM_CORES * NUM_SUBCORES
B_PER_W = BATCH // NW
CHUNK = 128
BT_PER_W = B_PER_W // CHUNK
N_CHUNKS = FIELDS * BT_PER_W
GROUPS = CHUNK // 16
DEPTH = 8
assert N_CHUNKS % DEPTH == 0 and DEPTH % 2 == 0

_T_BLK = 32768
_T_SUB = _T_BLK // 4
_T_GRID = (N_ROWS + _T_BLK - 1) // _T_BLK
_PACKED_ROWS = _T_GRID * _T_SUB


def _pack_body(x_ref, y_ref):
    x = x_ref[...]
    xs = jnp.concatenate(
        [x[:, k * _T_SUB:(k + 1) * _T_SUB] for k in range(4)], axis=0)
    y_ref[...] = xs.T


_pack_table = pl.pallas_call(
    _pack_body,
    grid=(_T_GRID,),
    in_specs=[pl.BlockSpec((OUT, _T_BLK), lambda i: (0, i))],
    out_specs=pl.BlockSpec((_T_SUB, 128), lambda i: (i, 0)),
    out_shape=jax.ShapeDtypeStruct((_PACKED_ROWS, 128), jnp.float32),
)


def _rsqrt(x):
    i = lax.bitcast_convert_type(x, jnp.int32)
    i = jnp.full((16,), 0x5F3759DF, jnp.int32) - (i >> 1)
    y = lax.bitcast_convert_type(i, jnp.float32)
    for _ in range(3):
        y = y * (1.5 - 0.5 * x * y * y)
    return y


_mesh = plsc.VectorSubcoreMesh(core_axis_name="c", subcore_axis_name="s")


@functools.partial(
    pl.kernel,
    out_type=jax.ShapeDtypeStruct(
        (FIELDS, OUT // 8, BATCH // 128, 8 * 128), jnp.float32),
    mesh=_mesh,
    scratch_types=[
        pltpu.VMEM((FIELDS, BT_PER_W, CHUNK), jnp.int32),
        [pltpu.VMEM((CHUNK, OUT), jnp.float32) for _ in range(DEPTH)],
        [pltpu.VMEM((OUT // 8, BT_PER_W, 8 * 128), jnp.float32)
         for _ in range(2)],
        [pltpu.SemaphoreType.DMA for _ in range(DEPTH)],
        [pltpu.SemaphoreType.DMA for _ in range(2)],
    ],
    compiler_params=pltpu.CompilerParams(
        needs_layout_passes=False, use_tc_tiling_on_sc=False
    ),
)
def _gather_normalize(table_hbm, idx_hbm, out_hbm,
                      idx_v, gbufs, obufs, gsems, wsems):
    wid = lax.axis_index("s") * NUM_CORES + lax.axis_index("c")

    lanes = lax.iota(jnp.int32, 16)

    def start_gather(f, bt, p):
        pltpu.async_copy(table_hbm.at[idx_v.at[f, bt]], gbufs[p], gsems[p])

    def wait_gather(f, bt, p):
        pltpu.make_async_copy(table_hbm.at[idx_v.at[f, bt]], gbufs[p],
                              gsems[p]).wait()

    def out_slice(f):
        return out_hbm.at[f, :, pl.ds(wid * BT_PER_W, BT_PER_W)]

    def start_write(f, q):
        pltpu.async_copy(obufs[q], out_slice(f), wsems[q])

    def wait_write(f, q):
        pltpu.make_async_copy(obufs[q], out_slice(f), wsems[q]).wait()

    def normalize(p, q, bt):
        src_v, dst_v = gbufs[p], obufs[q]
        bts = jnp.full((16,), bt, jnp.int32)

        def group_body(gr, c):
            row_ids = gr * 16 + lanes
            cols = [(lanes + j) & 31 for j in range(OUT)]
            elems = [plsc.load_gather(src_v, [row_ids, cols[j]])
                     for j in range(OUT)]
            acc = jnp.zeros((16,), jnp.float32)
            for e in elems:
                acc = acc + e * e
            inv = _rsqrt(acc)
            for j, e in enumerate(elems):
                plsc.store_scatter(
                    dst_v,
                    [cols[j] >> 3, bts, ((cols[j] & 7) << 7) + row_ids],
                    e * inv)
            return c

        lax.fori_loop(0, GROUPS, group_body, 0)

    pltpu.sync_copy(idx_hbm.at[wid], idx_v)
    for p in range(DEPTH):
        start_gather(p // BT_PER_W, p % BT_PER_W, p)

    def round_body(t, carry):
        for ff in range(2):
            f = t * 2 + ff
            q = ff

            @pl.when(f >= 2)
            def _():
                wait_write(f - 2, q)

            for bt in range(BT_PER_W):
                p = BT_PER_W * ff + bt
                wait_gather(f, bt, p)
                normalize(p, q, bt)

                @pl.when(f + 2 < FIELDS)
                def _():
                    start_gather(f + 2, bt, p)

            start_write(f, q)
        return carry

    lax.fori_loop(0, FIELDS // 2, round_body, 0)
    wait_write(FIELDS - 2, 0)
    wait_write(FIELDS - 1, 1)


def kernel(src, categories_means, categories_logvars):
    del categories_logvars
    packed = _pack_table(categories_means.T)
    table = packed.reshape(_PACKED_ROWS * 4, OUT)
    t = src.astype(jnp.int32)
    idx = (t & ~(_T_BLK - 1)) + 4 * (t & (_T_SUB - 1)) \
        + ((t >> _T_SUB.bit_length() - 1) & 3)
    idx = idx.reshape(NW, BT_PER_W, CHUNK, FIELDS).transpose(0, 3, 1, 2)
    x = _gather_normalize(table, idx)
    x = x.reshape(FIELDS, OUT // 8, BATCH // 128, 8, 128)
    return x.transpose(2, 4, 0, 1, 3).reshape(BATCH, FIELDS, OUT)
